# baseline (speedup 1.0000x reference)
import functools

import jax
import jax.numpy as jnp
from jax import lax
from jax.experimental import pallas as pl
from jax.experimental.pallas import tpu as pltpu
from jax.experimental.pallas import tpu_sc as plsc

_ROWS = 65536
_COLS = 512
_WORDS = _COLS // 4
_NC = 2
_NS = 16
_NW = _NC * _NS
_ROWS_PER_W = _ROWS // _NW
_BLK = 64
_GRP = _BLK // 16


def _sc_body(x_hbm, m_hbm, out_hbm, x_v, m_v, o_v):
    wid = lax.axis_index("s") * _NC + lax.axis_index("c")
    base = wid * _ROWS_PER_W

    def blk_body(b, carry_unused):
        r0 = base + b * _BLK
        pltpu.sync_copy(x_hbm.at[pl.ds(r0, _BLK)], x_v)
        pltpu.sync_copy(m_hbm.at[pl.ds(r0, _BLK)], m_v)

        def col_body(cw, carries):
            cvec = lax.broadcast(cw, (16,))
            new = []
            for g in range(_GRP):
                rows = lax.iota(jnp.int32, 16) + (16 * g)
                mw = plsc.load_gather(m_v, [rows, cvec])
                cg = carries[g]
                for bb in range(4):
                    col = cvec * 4 + bb
                    xv = plsc.load_gather(x_v, [rows, col])
                    bit = lax.shift_right_logical(mw, 8 * bb) & 1
                    cg = cg + jnp.where(bit != 0, xv, 0.0)
                    plsc.store_scatter(o_v, [rows, col], cg)
                new.append(cg)
            return tuple(new)

        zeros = tuple(jnp.zeros((16,), jnp.float32) for _ in range(_GRP))
        lax.fori_loop(0, _WORDS, col_body, zeros)
        pltpu.sync_copy(o_v, out_hbm.at[pl.ds(r0, _BLK)])
        return carry_unused

    lax.fori_loop(0, _ROWS_PER_W // _BLK, blk_body, 0)


_sc_call = functools.partial(
    pl.kernel,
    out_type=jax.ShapeDtypeStruct((_ROWS, _COLS), jnp.float32),
    mesh=plsc.VectorSubcoreMesh(core_axis_name="c", subcore_axis_name="s"),
    scratch_types=[
        pltpu.VMEM((_BLK, _COLS), jnp.float32),
        pltpu.VMEM((_BLK, _WORDS), jnp.int32),
        pltpu.VMEM((_BLK, _COLS), jnp.float32),
    ],
    compiler_params=pltpu.CompilerParams(needs_layout_passes=False),
)(_sc_body)


@jax.jit
def kernel(x, mask):
    m32 = jax.lax.bitcast_convert_type(
        mask.view(jnp.uint8).reshape(_ROWS, _WORDS, 4), jnp.int32)
    return _sc_call(x, m32)


# speedup vs baseline: 1.4038x; 1.4038x over previous
import functools

import jax
import jax.numpy as jnp
from jax import lax
from jax.experimental import pallas as pl
from jax.experimental.pallas import tpu as pltpu
from jax.experimental.pallas import tpu_sc as plsc

_ROWS = 65536
_COLS = 512
_WORDS = _COLS // 4
_NC = 2
_NS = 16
_NW = _NC * _NS
_ROWS_PER_W = _ROWS // _NW
_BLK = 64
_GRP = _BLK // 16


def _sc_body(x_hbm, m_hbm, out_hbm, x_v, m_v, o_v):
    wid = lax.axis_index("s") * _NC + lax.axis_index("c")
    base = wid * _ROWS_PER_W

    rows = [lax.iota(jnp.int32, 16) + (16 * g) for g in range(_GRP)]

    def blk_body(b, carry_unused):
        r0 = base + b * _BLK
        pltpu.sync_copy(x_hbm.at[pl.ds(r0, _BLK)], x_v)
        pltpu.sync_copy(m_hbm.at[pl.ds(r0, _BLK)], m_v)

        zeros = tuple(jnp.zeros((16,), jnp.float32) for _ in range(_GRP))

        @plsc.parallel_loop(0, _WORDS, unroll=8, carry=zeros)
        def _cols(cw, carries):
            cvec = lax.broadcast(cw, (16,))
            c4 = cvec * 4
            new = []
            for g in range(_GRP):
                mw = plsc.load_gather(m_v, [rows[g], cvec])
                cg = carries[g]
                for bb in range(4):
                    col = c4 + bb
                    xv = plsc.load_gather(x_v, [rows[g], col])
                    bit = lax.shift_right_logical(mw, 8 * bb) & 1
                    cg = cg + jnp.where(bit != 0, xv, 0.0)
                    plsc.store_scatter(o_v, [rows[g], col], cg)
                new.append(cg)
            return tuple(new)

        pltpu.sync_copy(o_v, out_hbm.at[pl.ds(r0, _BLK)])
        return carry_unused

    lax.fori_loop(0, _ROWS_PER_W // _BLK, blk_body, 0)


_sc_call = functools.partial(
    pl.kernel,
    out_type=jax.ShapeDtypeStruct((_ROWS, _COLS), jnp.float32),
    mesh=plsc.VectorSubcoreMesh(core_axis_name="c", subcore_axis_name="s"),
    scratch_types=[
        pltpu.VMEM((_BLK, _COLS), jnp.float32),
        pltpu.VMEM((_BLK, _WORDS), jnp.int32),
        pltpu.VMEM((_BLK, _COLS), jnp.float32),
    ],
    compiler_params=pltpu.CompilerParams(needs_layout_passes=False),
)(_sc_body)


@jax.jit
def kernel(x, mask):
    m32 = jax.lax.bitcast_convert_type(
        mask.view(jnp.uint8).reshape(_ROWS, _WORDS, 4), jnp.int32)
    return _sc_call(x, m32)


# speedup vs baseline: 1.4082x; 1.0031x over previous
import functools

import jax
import jax.numpy as jnp
from jax import lax
from jax.experimental import pallas as pl
from jax.experimental.pallas import tpu as pltpu
from jax.experimental.pallas import tpu_sc as plsc

_ROWS = 65536
_COLS = 512
_WORDS = _COLS // 4
_NC = 2
_NS = 16
_NW = _NC * _NS
_ROWS_PER_W = _ROWS // _NW
_BLK = 64
_GRP = _BLK // 16


def _sc_body(x_hbm, m_hbm, out_hbm, x_v, m_v, o_v):
    wid = lax.axis_index("s") * _NC + lax.axis_index("c")
    base = wid * _ROWS_PER_W

    rows = [lax.iota(jnp.int32, 16) + (16 * g) for g in range(_GRP)]

    def blk_body(b, carry_unused):
        r0 = base + b * _BLK
        pltpu.sync_copy(x_hbm.at[pl.ds(r0, _BLK)], x_v.at[:, pl.ds(0, _COLS)])
        pltpu.sync_copy(m_hbm.at[pl.ds(r0, _BLK)], m_v.at[:, pl.ds(0, _WORDS)])

        zeros = tuple(jnp.zeros((16,), jnp.float32) for _ in range(_GRP))

        @plsc.parallel_loop(0, _WORDS, unroll=8, carry=zeros)
        def _cols(cw, carries):
            cvec = lax.broadcast(cw, (16,))
            c4 = cvec * 4
            new = []
            for g in range(_GRP):
                mw = plsc.load_gather(m_v, [rows[g], cvec])
                cg = carries[g]
                for bb in range(4):
                    col = c4 + bb
                    xv = plsc.load_gather(x_v, [rows[g], col])
                    bit = lax.shift_right_logical(mw, 8 * bb) & 1
                    cg = cg + jnp.where(bit != 0, xv, 0.0)
                    plsc.store_scatter(o_v, [rows[g], col], cg)
                new.append(cg)
            return tuple(new)

        pltpu.sync_copy(o_v.at[:, pl.ds(0, _COLS)], out_hbm.at[pl.ds(r0, _BLK)])
        return carry_unused

    lax.fori_loop(0, _ROWS_PER_W // _BLK, blk_body, 0)


_sc_call = functools.partial(
    pl.kernel,
    out_type=jax.ShapeDtypeStruct((_ROWS, _COLS), jnp.float32),
    mesh=plsc.VectorSubcoreMesh(core_axis_name="c", subcore_axis_name="s"),
    scratch_types=[
        pltpu.VMEM((_BLK, _COLS + 1), jnp.float32),
        pltpu.VMEM((_BLK, _WORDS + 1), jnp.int32),
        pltpu.VMEM((_BLK, _COLS + 1), jnp.float32),
    ],
    compiler_params=pltpu.CompilerParams(needs_layout_passes=False),
)(_sc_body)


@jax.jit
def kernel(x, mask):
    m32 = jax.lax.bitcast_convert_type(
        mask.view(jnp.uint8).reshape(_ROWS, _WORDS, 4), jnp.int32)
    return _sc_call(x, m32)
